# baseline (speedup 1.0000x reference)
import jax
import jax.numpy as jnp
from jax import lax
from jax.experimental import pallas as pl
from jax.experimental.pallas import tpu as pltpu, tpu_sc as plsc

_NC, _NS = 2, 16
_NW = _NC * _NS
_CHUNK_ROWS = 224


def _sc_copy_body(in_hbm, out_hbm, buf0, buf1,
                  isem0, isem1, osem0, osem1):
    cid = lax.axis_index("c")
    sid = lax.axis_index("s")
    wid = sid * _NC + cid
    rows_per_w = in_hbm.shape[0] // _NW
    n = rows_per_w // _CHUNK_ROWS
    base = wid * rows_per_w
    bufs = (buf0, buf1)
    isems = (isem0, isem1)
    osems = (osem0, osem1)

    def region(ref, i):
        return ref.at[pl.ds(base + i * _CHUNK_ROWS, _CHUNK_ROWS)]

    copies_in = [None, None]
    copies_out = [None, None]
    copies_in[0] = pltpu.async_copy(region(in_hbm, 0), bufs[0], isems[0])
    for i in range(n):
        s = i % 2
        o = (i + 1) % 2
        if i + 1 < n:
            if i >= 1:
                copies_out[o].wait()
            copies_in[o] = pltpu.async_copy(region(in_hbm, i + 1),
                                            bufs[o], isems[o])
        copies_in[s].wait()
        copies_out[s] = pltpu.async_copy(bufs[s], region(out_hbm, i),
                                         osems[s])
    copies_out[(n - 1) % 2].wait()
    if n >= 2:
        copies_out[n % 2].wait()


def kernel(input):
    b, c, h, w = input.shape
    flat = jnp.transpose(input, (0, 2, 3, 1)).reshape(b * h * w, c)
    mesh = plsc.VectorSubcoreMesh(core_axis_name="c", subcore_axis_name="s",
                                  num_cores=_NC, num_subcores=_NS)
    run = pl.kernel(
        _sc_copy_body,
        out_type=jax.ShapeDtypeStruct(flat.shape, flat.dtype),
        mesh=mesh,
        scratch_types=[
            pltpu.VMEM((_CHUNK_ROWS, c), flat.dtype),
            pltpu.VMEM((_CHUNK_ROWS, c), flat.dtype),
            pltpu.SemaphoreType.DMA,
            pltpu.SemaphoreType.DMA,
            pltpu.SemaphoreType.DMA,
            pltpu.SemaphoreType.DMA,
        ],
        compiler_params=pltpu.CompilerParams(use_tc_tiling_on_sc=True),
    )
    out = run(flat)
    return jnp.transpose(out.reshape(b, h, w, c), (0, 3, 1, 2))


# speedup vs baseline: 1.4142x; 1.4142x over previous
import jax
import jax.numpy as jnp
from jax.experimental import pallas as pl
from jax.experimental.pallas import tpu as pltpu

_CHUNK_ROWS = 6272
_DEPTH = 4
_LOOK = 2


def _copy_body(in_hbm, out_hbm, bufs, isems, osems):
    n = in_hbm.shape[0] // _CHUNK_ROWS

    def region(ref, i):
        return ref.at[pl.ds(i * _CHUNK_ROWS, _CHUNK_ROWS)]

    def in_copy(j, s):
        return pltpu.make_async_copy(region(in_hbm, j), bufs.at[s], isems.at[s])

    def out_copy(i, s):
        return pltpu.make_async_copy(bufs.at[s], region(out_hbm, i), osems.at[s])

    for j in range(_LOOK):
        in_copy(j, j % _DEPTH).start()
    for i in range(n):
        s = i % _DEPTH
        in_copy(i, s).wait()
        out_copy(i, s).start()
        j = i + _LOOK
        if j < n:
            sj = j % _DEPTH
            if j >= _DEPTH:
                out_copy(j - _DEPTH, sj).wait()
            in_copy(j, sj).start()
    for k in range(max(n - _DEPTH, 0), n):
        out_copy(k, k % _DEPTH).wait()


def kernel(input):
    b, c, h, w = input.shape
    flat = jnp.transpose(input, (0, 2, 3, 1)).reshape(b * h * w, c)
    out = pl.pallas_call(
        _copy_body,
        out_shape=jax.ShapeDtypeStruct(flat.shape, flat.dtype),
        in_specs=[pl.BlockSpec(memory_space=pl.ANY)],
        out_specs=pl.BlockSpec(memory_space=pl.ANY),
        scratch_shapes=[
            pltpu.VMEM((_DEPTH, _CHUNK_ROWS, c), flat.dtype),
            pltpu.SemaphoreType.DMA((_DEPTH,)),
            pltpu.SemaphoreType.DMA((_DEPTH,)),
        ],
    )(flat)
    return jnp.transpose(out.reshape(b, h, w, c), (0, 3, 1, 2))


# speedup vs baseline: 1.4166x; 1.0018x over previous
import jax
import jax.numpy as jnp
from jax.experimental import pallas as pl
from jax.experimental.pallas import tpu as pltpu

_BLOCK_ROWS = 14336


def _copy_body(in_ref, out_ref):
    out_ref[...] = in_ref[...]


def kernel(input):
    b, c, h, w = input.shape
    flat = jnp.transpose(input, (0, 2, 3, 1)).reshape(b * h * w, c)
    rows = flat.shape[0]
    out = pl.pallas_call(
        _copy_body,
        out_shape=jax.ShapeDtypeStruct(flat.shape, flat.dtype),
        grid=(rows // _BLOCK_ROWS,),
        in_specs=[pl.BlockSpec((_BLOCK_ROWS, c), lambda i: (i, 0))],
        out_specs=pl.BlockSpec((_BLOCK_ROWS, c), lambda i: (i, 0)),
    )(flat)
    return jnp.transpose(out.reshape(b, h, w, c), (0, 3, 1, 2))
